# baseline (speedup 1.0000x reference)
import functools

import jax
import jax.numpy as jnp
import numpy as np
from jax import lax
from jax.experimental import pallas as pl
from jax.experimental.pallas import tpu as pltpu
from jax.experimental.pallas import tpu_sc as plsc

_NUM_MOVES = 1968
_NEG = np.float32(-1e30)


def _threefry2x32(x0, x1):
    k0 = jnp.uint32(0)
    k1 = jnp.uint32(1)
    ks = [k0, k1, k0 ^ k1 ^ jnp.uint32(0x1BD11BDA)]
    rot_a = [13, 15, 26, 6]
    rot_b = [17, 29, 16, 24]

    def rotl(x, r):
        return (x << jnp.uint32(r)) | (x >> jnp.uint32(32 - r))

    x0 = x0 + ks[0]
    x1 = x1 + ks[1]
    for i, rots in enumerate([rot_a, rot_b, rot_a, rot_b, rot_a]):
        for r in rots:
            x0 = x0 + x1
            x1 = rotl(x1, r)
            x1 = x0 ^ x1
        x0 = x0 + ks[(i + 1) % 3]
        x1 = x1 + ks[(i + 2) % 3] + jnp.uint32(i + 1)
    return x0, x1


def _gumbel_from_flat_idx(flat_idx):
    i = flat_idx.astype(jnp.uint32)
    z0, z1 = _threefry2x32(jnp.zeros_like(i), i)
    bits = z0 ^ z1
    f = lax.bitcast_convert_type(
        (bits >> jnp.uint32(9)) | jnp.uint32(0x3F800000), jnp.float32
    ) - jnp.float32(1.0)
    span = np.float32(1.0) - np.float32(1e-10)
    u = jnp.maximum(jnp.float32(1e-10), f * span + jnp.float32(1e-10))
    return -jnp.log(-jnp.log(u))


def _sc_gather_mask(logits, flat_ids):
    n = flat_ids.shape[0]
    info = plsc.get_sparse_core_info()
    nw = info.num_cores * info.num_subcores
    per = n // nw
    rows_per = per // 64
    mesh = plsc.VectorSubcoreMesh(core_axis_name="c", subcore_axis_name="s")

    @functools.partial(
        pl.kernel,
        mesh=mesh,
        compiler_params=pltpu.CompilerParams(needs_layout_passes=False),
        out_type=jax.ShapeDtypeStruct((n,), jnp.float32),
        scratch_types=[
            pltpu.VMEM((_NUM_MOVES,), jnp.float32),
            pltpu.VMEM((_NUM_MOVES,), jnp.int32),
            pltpu.VMEM((_NUM_MOVES,), jnp.int32),
            pltpu.VMEM((_NUM_MOVES,), jnp.int32),
            pltpu.VMEM((_NUM_MOVES,), jnp.int32),
            pltpu.VMEM((per,), jnp.int32),
            pltpu.VMEM((per,), jnp.float32),
        ],
    )
    def sc_kernel(logits_hbm, ids_hbm, gm_hbm, table_v, slot_v, slot_v2,
                  slot_v3, slot_v4, idx_v, gm_v):
        wid = lax.axis_index("s") * info.num_cores + lax.axis_index("c")
        base = wid * per
        pltpu.sync_copy(logits_hbm, table_v)
        pltpu.sync_copy(ids_hbm.at[pl.ds(base, per)], idx_v)
        lane = lax.iota(jnp.int32, 16)

        def row_body(r4, carry):
            rb = pl.multiple_of(r4 * 256, 256)
            for j, slot_t in enumerate((slot_v, slot_v2, slot_v3, slot_v4)):
                idxs = []
                gs = []
                for k in range(4):
                    sl = pl.ds(rb + j * 64 + k * 16, 16)
                    idx = idx_v[sl]
                    idxs.append(idx)
                    gs.append(plsc.load_gather(table_v, [idx]))
                    plsc.store_scatter(slot_t, [idx], lane + jnp.int32(k * 16))
                m01 = jnp.maximum(gs[0], gs[1])
                m23 = jnp.maximum(gs[2], gs[3])
                m = lax.reduce_max(jnp.maximum(m01, m23), axes=(0,))
                mv = jnp.full((16,), m, dtype=jnp.float32)
                for k in range(4):
                    sl = pl.ds(rb + j * 64 + k * 16, 16)
                    winner = plsc.load_gather(slot_t, [idxs[k]])
                    gm_v[sl] = jnp.where(
                        winner == lane + jnp.int32(k * 16), gs[k] - mv, _NEG)
            return carry

        lax.fori_loop(0, rows_per // 4, row_body, 0)
        pltpu.sync_copy(gm_v, gm_hbm.at[pl.ds(base, per)])

    return sc_kernel(logits, flat_ids)


def _tc_gumbel(ids_wide, half_b, block_rows):
    n, w = ids_wide.shape
    grid = (n // block_rows,)

    def body(ids_ref, gum_ref):
        ids = ids_ref[...]
        wrow = pl.program_id(0) * block_rows + lax.broadcasted_iota(
            jnp.int32, (block_rows, w), 0)
        lanes = lax.broadcasted_iota(jnp.int32, (block_rows, w), 1)
        row = wrow + jnp.where(lanes >= 64, jnp.int32(half_b), jnp.int32(0))
        gum_ref[...] = _gumbel_from_flat_idx(row * jnp.int32(_NUM_MOVES) + ids)

    return pl.pallas_call(
        body,
        grid=grid,
        in_specs=[pl.BlockSpec((block_rows, w), lambda i: (i, 0))],
        out_specs=pl.BlockSpec((block_rows, w), lambda i: (i, 0)),
        out_shape=jax.ShapeDtypeStruct((n, w), jnp.float32),
    )(ids_wide)


def _tc_combine(ids_wide, gm_wide, gum_wide, b, l, wide_block):
    nw = b * l // 128
    grid = (nw // wide_block,)
    w = wide_block

    ng = nw // w

    def body(ids_ref, gm_ref, gum_ref, se_ref, so_ref, le_ref, lo_ref):
        idsf = ids_ref[...].astype(jnp.float32)

        def seg(x, red):
            a = red(x[:, :64], axis=1, keepdims=True)
            c = red(x[:, 64:], axis=1, keepdims=True)
            return jnp.concatenate(
                [jnp.broadcast_to(a, (w, 64)), jnp.broadcast_to(c, (w, 64))],
                axis=1)

        e = jnp.exp(gm_ref[...])
        hi = lax.shift_right_logical(
            lax.broadcasted_iota(jnp.int32, (128, 128), 0), 6)
        hj = lax.shift_right_logical(
            lax.broadcasted_iota(jnp.int32, (128, 128), 1), 6)
        ones_bd = jnp.where(hi == hj, jnp.float32(1.0), jnp.float32(0.0))
        z = lax.dot_general(e, ones_bd, (((1,), (0,)), ((), ())),
                            precision=lax.Precision.HIGHEST,
                            preferred_element_type=jnp.float32)
        logp = jnp.log(e / z + jnp.float32(1e-30))
        cand = logp + gum_ref[...]
        maxv = seg(cand, jnp.max)
        wids = jnp.where(cand == maxv, idsf, jnp.float32(3e38))
        sa = jnp.min(wids[:, :64], axis=1)
        sc = jnp.min(wids[:, 64:], axis=1)
        se_ref[...] = sa.astype(jnp.int32)
        so_ref[...] = sc.astype(jnp.int32)
        samp = jnp.concatenate(
            [jnp.broadcast_to(sa[:, None], (w, 64)),
             jnp.broadcast_to(sc[:, None], (w, 64))], axis=1)
        lp = jnp.where(idsf == samp, logp, jnp.float32(-3e38))
        le_ref[...] = jnp.max(lp[:, :64], axis=1)
        lo_ref[...] = jnp.max(lp[:, 64:], axis=1)

    return pl.pallas_call(
        body,
        grid=grid,
        in_specs=[
            pl.BlockSpec((w, 128), lambda i: (i, 0)),
            pl.BlockSpec((w, 128), lambda i: (i, 0)),
            pl.BlockSpec((w, 128), lambda i: (i, 0)),
        ],
        out_specs=[
            pl.BlockSpec((w,), lambda i: (i,)),
            pl.BlockSpec((w,), lambda i: (i,)),
            pl.BlockSpec((w,), lambda i: (i,)),
            pl.BlockSpec((w,), lambda i: (i,)),
        ],
        out_shape=[
            jax.ShapeDtypeStruct((nw,), jnp.int32),
            jax.ShapeDtypeStruct((nw,), jnp.int32),
            jax.ShapeDtypeStruct((nw,), jnp.float32),
            jax.ShapeDtypeStruct((nw,), jnp.float32),
        ],
    )(ids_wide, gm_wide, gum_wide)


def kernel(legal_ids, logits):
    b, l = legal_ids.shape
    nw = b * l // 128
    ids_wide = jnp.concatenate([legal_ids[: b // 2], legal_ids[b // 2:]],
                               axis=1)
    flat_ids = ids_wide.reshape(-1)
    gm_flat = _sc_gather_mask(logits, flat_ids)
    gum_wide = _tc_gumbel(ids_wide, b // 2, 512)
    s_lo, s_hi, l_lo, l_hi = _tc_combine(
        ids_wide, gm_flat.reshape(nw, 128), gum_wide, b, l, 512)
    sample = jnp.concatenate([s_lo.reshape(-1), s_hi.reshape(-1)])
    logp = jnp.concatenate([l_lo.reshape(-1), l_hi.reshape(-1)]).reshape(b, 1)
    return sample, logp


# speedup vs baseline: 1.0582x; 1.0582x over previous
import functools

import jax
import jax.numpy as jnp
import numpy as np
from jax import lax
from jax.experimental import pallas as pl
from jax.experimental.pallas import tpu as pltpu
from jax.experimental.pallas import tpu_sc as plsc

_NUM_MOVES = 1968
_NEG = np.float32(-1e30)


def _threefry2x32(x0, x1):
    k0 = jnp.uint32(0)
    k1 = jnp.uint32(1)
    ks = [k0, k1, k0 ^ k1 ^ jnp.uint32(0x1BD11BDA)]
    rot_a = [13, 15, 26, 6]
    rot_b = [17, 29, 16, 24]

    def rotl(x, r):
        return (x << jnp.uint32(r)) | (x >> jnp.uint32(32 - r))

    x0 = x0 + ks[0]
    x1 = x1 + ks[1]
    for i, rots in enumerate([rot_a, rot_b, rot_a, rot_b, rot_a]):
        for r in rots:
            x0 = x0 + x1
            x1 = rotl(x1, r)
            x1 = x0 ^ x1
        x0 = x0 + ks[(i + 1) % 3]
        x1 = x1 + ks[(i + 2) % 3] + jnp.uint32(i + 1)
    return x0, x1


def _gumbel_from_flat_idx(flat_idx):
    i = flat_idx.astype(jnp.uint32)
    z0, z1 = _threefry2x32(jnp.zeros_like(i), i)
    bits = z0 ^ z1
    f = lax.bitcast_convert_type(
        (bits >> jnp.uint32(9)) | jnp.uint32(0x3F800000), jnp.float32
    ) - jnp.float32(1.0)
    span = np.float32(1.0) - np.float32(1e-10)
    u = jnp.maximum(jnp.float32(1e-10), f * span + jnp.float32(1e-10))
    return -jnp.log(-jnp.log(u))


def _sc_gather_mask(logits, ids_wide):
    nwide, wl = ids_wide.shape
    info = plsc.get_sparse_core_info()
    nworker = info.num_cores * info.num_subcores
    wper = nwide // nworker
    mesh = plsc.VectorSubcoreMesh(core_axis_name="c", subcore_axis_name="s")

    @functools.partial(
        pl.kernel,
        mesh=mesh,
        compiler_params=pltpu.CompilerParams(needs_layout_passes=False),
        out_type=jax.ShapeDtypeStruct((nwide, wl), jnp.float32),
        scratch_types=(
            [pltpu.VMEM((_NUM_MOVES,), jnp.float32)]
            + [pltpu.VMEM((_NUM_MOVES,), jnp.int32) for _ in range(8)]
            + [pltpu.VMEM((wper, wl), jnp.int32),
               pltpu.VMEM((wper, wl), jnp.float32)]
        ),
    )
    def sc_kernel(logits_hbm, ids_hbm, gm_hbm, table_v, s0, s1, s2, s3, s4,
                  s5, s6, s7, idx_v, gm_v):
        slot_tables = (s0, s1, s2, s3, s4, s5, s6, s7)
        wid = lax.axis_index("s") * info.num_cores + lax.axis_index("c")
        base = wid * wper
        pltpu.sync_copy(logits_hbm, table_v)
        pltpu.sync_copy(ids_hbm.at[pl.ds(base, wper)], idx_v)
        lane = lax.iota(jnp.int32, 16)

        def row_body(it, carry):
            wr0 = pl.multiple_of(it * 4, 4)
            for j in range(8):
                wr = wr0 + j // 2
                cb = (j % 2) * 64
                slot_t = slot_tables[j]
                idxs = []
                gs = []
                for k in range(4):
                    sl = pl.ds(cb + k * 16, 16)
                    idx = idx_v[wr, sl]
                    idxs.append(idx)
                    gs.append(plsc.load_gather(table_v, [idx]))
                    plsc.store_scatter(slot_t, [idx], lane + jnp.int32(k * 16))
                m01 = jnp.maximum(gs[0], gs[1])
                m23 = jnp.maximum(gs[2], gs[3])
                m = lax.reduce_max(jnp.maximum(m01, m23), axes=(0,))
                mv = jnp.full((16,), m, dtype=jnp.float32)
                for k in range(4):
                    sl = pl.ds(cb + k * 16, 16)
                    winner = plsc.load_gather(slot_t, [idxs[k]])
                    gm_v[wr, sl] = jnp.where(
                        winner == lane + jnp.int32(k * 16), gs[k] - mv, _NEG)
            return carry

        lax.fori_loop(0, wper // 4, row_body, 0)
        pltpu.sync_copy(gm_v, gm_hbm.at[pl.ds(base, wper)])

    return sc_kernel(logits, ids_wide)


def _tc_gumbel(ids_wide, half_b, block_rows):
    n, w = ids_wide.shape
    grid = (n // block_rows,)

    def body(ids_ref, gum_ref):
        ids = ids_ref[...]
        wrow = pl.program_id(0) * block_rows + lax.broadcasted_iota(
            jnp.int32, (block_rows, w), 0)
        lanes = lax.broadcasted_iota(jnp.int32, (block_rows, w), 1)
        row = wrow + jnp.where(lanes >= 64, jnp.int32(half_b), jnp.int32(0))
        gum_ref[...] = _gumbel_from_flat_idx(row * jnp.int32(_NUM_MOVES) + ids)

    return pl.pallas_call(
        body,
        grid=grid,
        in_specs=[pl.BlockSpec((block_rows, w), lambda i: (i, 0))],
        out_specs=pl.BlockSpec((block_rows, w), lambda i: (i, 0)),
        out_shape=jax.ShapeDtypeStruct((n, w), jnp.float32),
    )(ids_wide)


def _tc_combine(ids_wide, gm_wide, gum_wide, b, l, wide_block):
    nw = b * l // 128
    grid = (nw // wide_block,)
    w = wide_block

    ng = nw // w

    def body(ids_ref, gm_ref, gum_ref, se_ref, so_ref, le_ref, lo_ref):
        idsf = ids_ref[...].astype(jnp.float32)

        def seg(x, red):
            a = red(x[:, :64], axis=1, keepdims=True)
            c = red(x[:, 64:], axis=1, keepdims=True)
            return jnp.concatenate(
                [jnp.broadcast_to(a, (w, 64)), jnp.broadcast_to(c, (w, 64))],
                axis=1)

        e = jnp.exp(gm_ref[...])
        z = seg(e, jnp.sum)
        logp = jnp.log(e / z + jnp.float32(1e-30))
        cand = logp + gum_ref[...]
        maxv = seg(cand, jnp.max)
        wids = jnp.where(cand == maxv, idsf, jnp.float32(3e38))
        sa = jnp.min(wids[:, :64], axis=1)
        sc = jnp.min(wids[:, 64:], axis=1)
        se_ref[...] = sa.astype(jnp.int32)
        so_ref[...] = sc.astype(jnp.int32)
        samp = jnp.concatenate(
            [jnp.broadcast_to(sa[:, None], (w, 64)),
             jnp.broadcast_to(sc[:, None], (w, 64))], axis=1)
        lp = jnp.where(idsf == samp, logp, jnp.float32(-3e38))
        le_ref[...] = jnp.max(lp[:, :64], axis=1)
        lo_ref[...] = jnp.max(lp[:, 64:], axis=1)

    return pl.pallas_call(
        body,
        grid=grid,
        in_specs=[
            pl.BlockSpec((w, 128), lambda i: (i, 0)),
            pl.BlockSpec((w, 128), lambda i: (i, 0)),
            pl.BlockSpec((w, 128), lambda i: (i, 0)),
        ],
        out_specs=[
            pl.BlockSpec((w,), lambda i: (i,)),
            pl.BlockSpec((w,), lambda i: (i,)),
            pl.BlockSpec((w,), lambda i: (i,)),
            pl.BlockSpec((w,), lambda i: (i,)),
        ],
        out_shape=[
            jax.ShapeDtypeStruct((nw,), jnp.int32),
            jax.ShapeDtypeStruct((nw,), jnp.int32),
            jax.ShapeDtypeStruct((nw,), jnp.float32),
            jax.ShapeDtypeStruct((nw,), jnp.float32),
        ],
    )(ids_wide, gm_wide, gum_wide)


def kernel(legal_ids, logits):
    b, l = legal_ids.shape
    nw = b * l // 128
    ids_wide = jnp.concatenate([legal_ids[: b // 2], legal_ids[b // 2:]],
                               axis=1)
    gm_wide = _sc_gather_mask(logits, ids_wide)
    gum_wide = _tc_gumbel(ids_wide, b // 2, 512)
    s_lo, s_hi, l_lo, l_hi = _tc_combine(
        ids_wide, gm_wide, gum_wide, b, l, 512)
    sample = jnp.concatenate([s_lo.reshape(-1), s_hi.reshape(-1)])
    logp = jnp.concatenate([l_lo.reshape(-1), l_hi.reshape(-1)]).reshape(b, 1)
    return sample, logp
